# baseline (speedup 1.0000x reference)
import functools

import jax
import jax.numpy as jnp
from jax import lax
from jax.experimental import pallas as pl
from jax.experimental.pallas import tpu as pltpu
from jax.experimental.pallas import tpu_sc as plsc


def kernel(x, pos_embedding):
    S, N = x.shape
    _, D = pos_embedding.shape

    info = plsc.get_sparse_core_info()
    NW = info.num_cores * info.num_subcores
    rows_per_w = S // NW
    BS = 32
    NBUF = 3
    n_chunks = rows_per_w // BS

    mesh = plsc.VectorSubcoreMesh(core_axis_name="c", subcore_axis_name="s")

    @functools.partial(
        pl.kernel,
        out_type=jax.ShapeDtypeStruct((S, N * D), jnp.float32),
        mesh=mesh,
        scratch_types=(
            [pltpu.VMEM((BS, D), jnp.float32)] * NBUF
            + [pltpu.SemaphoreType.DMA] * (2 * NBUF)
        ),
    )
    def body(table_hbm, out_hbm, *scr):
        bufs = scr[:NBUF]
        rsems = scr[NBUF:2 * NBUF]
        wsems = scr[2 * NBUF:]
        wid = lax.axis_index("s") * info.num_cores + lax.axis_index("c")
        base0 = wid * rows_per_w

        read_h = [None] * n_chunks
        write_h = [[] for _ in range(n_chunks)]
        for c in range(min(NBUF, n_chunks)):
            read_h[c] = pltpu.async_copy(
                table_hbm.at[pl.ds(base0 + c * BS, BS)], bufs[c], rsems[c])
        for c in range(n_chunks):
            b = c % NBUF
            read_h[c].wait()
            for n in range(N):
                write_h[c].append(pltpu.async_copy(
                    bufs[b],
                    out_hbm.at[pl.ds(base0 + c * BS, BS), pl.ds(n * D, D)],
                    wsems[b]))
            nxt = c + NBUF
            if nxt < n_chunks:
                for h in write_h[c]:
                    h.wait()
                read_h[nxt] = pltpu.async_copy(
                    table_hbm.at[pl.ds(base0 + nxt * BS, BS)], bufs[b], rsems[b])
        for c in range(max(0, n_chunks - NBUF), n_chunks):
            for h in write_h[c]:
                h.wait()

    return body(pos_embedding).reshape(S, N, D)


# speedup vs baseline: 3.8144x; 3.8144x over previous
import functools

import jax
import jax.numpy as jnp
from jax.experimental import pallas as pl
from jax.experimental.pallas import tpu as pltpu


def kernel(x, pos_embedding):
    S, N = x.shape
    _, D = pos_embedding.shape
    BS = 256

    def body(tab_ref, out_ref):
        out_ref[...] = jnp.broadcast_to(tab_ref[...][:, None, :], (BS, N, D))

    out = pl.pallas_call(
        body,
        grid=(S // BS,),
        in_specs=[pl.BlockSpec((BS, D), lambda i: (i, 0))],
        out_specs=pl.BlockSpec((BS, N, D), lambda i: (i, 0, 0)),
        out_shape=jax.ShapeDtypeStruct((S, N, D), jnp.float32),
    )(pos_embedding)
    return out
